# baseline (speedup 1.0000x reference)
import functools

import jax
import jax.numpy as jnp
from jax import lax
from jax.experimental import pallas as pl
from jax.experimental.pallas import tpu as pltpu
from jax.experimental.pallas import tpu_sc as plsc

VOCAB = 1_000_000
DIM = 64
SEQ = 50
NROW = 16384

NC = 2
NS = 16
NW = NC * NS

NPW = NROW // NW
NL = 256
NB = NPW // NL
K = SEQ * NB
NBUF = 4
OUTER = K // NBUF

_BT = 512
_TGRID = (VOCAB + _BT - 1) // _BT


def _tt_body(a_ref, o_ref):
    a = a_ref[...]
    e = jnp.eye(DIM, dtype=jnp.float32)
    t = lax.dot_general(a, e, (((0,), (0,)), ((), ())),
                        preferred_element_type=jnp.float32)
    ep = jnp.concatenate([e, jnp.zeros((DIM, DIM), jnp.float32)], axis=1)
    o_ref[...] = lax.dot_general(t, ep, (((1,), (0,)), ((), ())),
                                 preferred_element_type=jnp.float32)


_tc_table = pl.pallas_call(
    _tt_body,
    grid=(_TGRID,),
    in_specs=[pl.BlockSpec((DIM, _BT), lambda j: (0, j))],
    out_specs=pl.BlockSpec((_BT, 2 * DIM), lambda j: (j, 0)),
    out_shape=jax.ShapeDtypeStruct((VOCAB, 2 * DIM), jnp.float32),
)

_mesh = plsc.VectorSubcoreMesh(
    core_axis_name="c", subcore_axis_name="s", num_cores=NC, num_subcores=NS
)


@functools.partial(
    pl.kernel,
    out_type=jax.ShapeDtypeStruct((SEQ * NROW, 2 * DIM), jnp.float32),
    mesh=_mesh,
    scratch_types=[
        pltpu.VMEM((SEQ, NPW), jnp.int32),
        pltpu.VMEM((NBUF, NL, DIM), jnp.float32),
        [pltpu.SemaphoreType.DMA] * NBUF,
    ],
    compiler_params=pltpu.CompilerParams(use_tc_tiling_on_sc=False),
)
def _emb_gather(xt_hbm, table_hbm, out_hbm, idx_v, rows_v, gsems):
    wid = lax.axis_index("s") * NC + lax.axis_index("c")
    nbase = wid * NPW
    pltpu.sync_copy(xt_hbm.at[:, pl.ds(nbase, NPW)], idx_v)

    def fire(kk, b):
        s = kk // NB
        nb = kk % NB
        pltpu.async_copy(table_hbm.at[idx_v.at[s, pl.ds(nb * NL, NL)]],
                         rows_v.at[b], gsems[b])

    def drain(kk, b):
        s = kk // NB
        nb = kk % NB
        pltpu.make_async_copy(table_hbm.at[idx_v.at[s, pl.ds(nb * NL, NL)]],
                              rows_v.at[b], gsems[b]).wait()
        pltpu.sync_copy(
            rows_v.at[b],
            out_hbm.at[pl.ds(s * NROW + nbase + nb * NL, NL), pl.ds(0, DIM)])

    for b in range(NBUF):
        fire(b, b)

    @pl.loop(0, OUTER - 1)
    def _outer(o):
        for b in range(NBUF):
            kk = o * NBUF + b
            drain(kk, b)
            fire(kk + NBUF, b)

    for b in range(NBUF):
        drain((OUTER - 1) * NBUF + b, b)


_TN = 512


def _fmt_body(g_ref, o_ref):
    u = g_ref[:, :DIM]
    e = jnp.eye(DIM, dtype=jnp.float32)
    o_ref[0] = lax.dot_general(e, u, (((1,), (1,)), ((), ())),
                               preferred_element_type=jnp.float32)


_tc_format = pl.pallas_call(
    _fmt_body,
    grid=(SEQ, NROW // _TN),
    in_specs=[pl.BlockSpec((_TN, 2 * DIM),
                           lambda s, nb: (s * (NROW // _TN) + nb, 0))],
    out_specs=pl.BlockSpec((1, DIM, _TN), lambda s, nb: (s, 0, nb)),
    out_shape=jax.ShapeDtypeStruct((SEQ, DIM, NROW), jnp.float32),
)


def kernel(x, emb):
    t2 = _tc_table(emb.T)
    table = t2.reshape(2 * VOCAB, DIM)
    g2 = _emb_gather((x * 2).T, table)
    ol = _tc_format(g2)
    return ol.transpose(2, 0, 1)


# speedup vs baseline: 1.2070x; 1.2070x over previous
import functools

import jax
import jax.numpy as jnp
from jax import lax
from jax.experimental import pallas as pl
from jax.experimental.pallas import tpu as pltpu
from jax.experimental.pallas import tpu_sc as plsc

VOCAB = 1_000_000
DIM = 64
SEQ = 50
NROW = 16384

NC = 2
NS = 16
NW = NC * NS

NPW = NROW // NW
NL = 256
NB = NPW // NL
K = SEQ * NB
NBUF = 4
OUTER = K // NBUF

_BT = 8192
_TGRID = (VOCAB + _BT - 1) // _BT


def _tt_body(a_ref, o_ref):
    a = a_ref[...]
    e = jnp.eye(DIM, dtype=jnp.float32)
    t = lax.dot_general(a, e, (((0,), (0,)), ((), ())),
                        preferred_element_type=jnp.float32,
                        precision=lax.Precision.HIGHEST)
    ep = jnp.concatenate([e, jnp.zeros((DIM, DIM), jnp.float32)], axis=1)
    o_ref[...] = lax.dot_general(t, ep, (((1,), (0,)), ((), ())),
                                 preferred_element_type=jnp.float32,
                                 precision=lax.Precision.HIGHEST)


_tc_table = pl.pallas_call(
    _tt_body,
    grid=(_TGRID,),
    in_specs=[pl.BlockSpec((DIM, _BT), lambda j: (0, j))],
    out_specs=pl.BlockSpec((_BT, 2 * DIM), lambda j: (j, 0)),
    out_shape=jax.ShapeDtypeStruct((VOCAB, 2 * DIM), jnp.float32),
)

_mesh = plsc.VectorSubcoreMesh(
    core_axis_name="c", subcore_axis_name="s", num_cores=NC, num_subcores=NS
)


@functools.partial(
    pl.kernel,
    out_type=jax.ShapeDtypeStruct((SEQ * NROW, 2 * DIM), jnp.float32),
    mesh=_mesh,
    scratch_types=[
        pltpu.VMEM((SEQ, NPW), jnp.int32),
        pltpu.VMEM((NBUF, NL, DIM), jnp.float32),
        [pltpu.SemaphoreType.DMA] * NBUF,
    ],
    compiler_params=pltpu.CompilerParams(use_tc_tiling_on_sc=False),
)
def _emb_gather(xt_hbm, table_hbm, out_hbm, idx_v, rows_v, gsems):
    wid = lax.axis_index("s") * NC + lax.axis_index("c")
    nbase = wid * NPW
    pltpu.sync_copy(xt_hbm.at[:, pl.ds(nbase, NPW)], idx_v)

    def fire(kk, b):
        s = kk // NB
        nb = kk % NB
        pltpu.async_copy(table_hbm.at[idx_v.at[s, pl.ds(nb * NL, NL)]],
                         rows_v.at[b], gsems[b])

    def drain(kk, b):
        s = kk // NB
        nb = kk % NB
        pltpu.make_async_copy(table_hbm.at[idx_v.at[s, pl.ds(nb * NL, NL)]],
                              rows_v.at[b], gsems[b]).wait()
        pltpu.sync_copy(
            rows_v.at[b],
            out_hbm.at[pl.ds(s * NROW + nbase + nb * NL, NL), pl.ds(0, DIM)])

    for b in range(NBUF):
        fire(b, b)

    @pl.loop(0, OUTER - 1)
    def _outer(o):
        for b in range(NBUF):
            kk = o * NBUF + b
            drain(kk, b)
            fire(kk + NBUF, b)

    for b in range(NBUF):
        drain((OUTER - 1) * NBUF + b, b)


_TN = 8192


def _fmt_body(g_ref, o_ref):
    u = g_ref[:, :DIM]
    e = jnp.eye(DIM, dtype=jnp.float32)
    o_ref[0] = lax.dot_general(e, u, (((1,), (1,)), ((), ())),
                               preferred_element_type=jnp.float32,
                               precision=lax.Precision.HIGHEST)


_tc_format = pl.pallas_call(
    _fmt_body,
    grid=(SEQ, NROW // _TN),
    in_specs=[pl.BlockSpec((_TN, 2 * DIM),
                           lambda s, nb: (s * (NROW // _TN) + nb, 0))],
    out_specs=pl.BlockSpec((1, DIM, _TN), lambda s, nb: (s, 0, nb)),
    out_shape=jax.ShapeDtypeStruct((SEQ, DIM, NROW), jnp.float32),
)


def kernel(x, emb):
    t2 = _tc_table(emb.T)
    table = t2.reshape(2 * VOCAB, DIM)
    g2 = _emb_gather((x * 2).T, table)
    ol = _tc_format(g2)
    return ol.transpose(2, 0, 1)


# speedup vs baseline: 1.2640x; 1.0472x over previous
import functools

import jax
import jax.numpy as jnp
from jax import lax
from jax.experimental import pallas as pl
from jax.experimental.pallas import tpu as pltpu
from jax.experimental.pallas import tpu_sc as plsc

VOCAB = 1_000_000
DIM = 64
SEQ = 50
NROW = 16384

NC = 2
NS = 16
NW = NC * NS

NPW = NROW // NW
NL = 256
NB = NPW // NL
K = SEQ * NB
NBUF = 4
OUTER = K // NBUF

_BT = 8192
_TGRID = (VOCAB + _BT - 1) // _BT


def _tt_body(a_ref, o_ref):
    a = a_ref[...]
    e = jnp.eye(DIM, dtype=jnp.float32)
    t = lax.dot_general(a, e, (((0,), (0,)), ((), ())),
                        preferred_element_type=jnp.float32,
                        precision=lax.Precision.HIGHEST)
    ep = jnp.concatenate([e, jnp.zeros((DIM, DIM), jnp.float32)], axis=1)
    o_ref[...] = lax.dot_general(t, ep, (((1,), (0,)), ((), ())),
                                 preferred_element_type=jnp.float32,
                                 precision=lax.Precision.HIGHEST)


_tc_table = pl.pallas_call(
    _tt_body,
    grid=(_TGRID,),
    in_specs=[pl.BlockSpec((DIM, _BT), lambda j: (0, j))],
    out_specs=pl.BlockSpec((_BT, 2 * DIM), lambda j: (j, 0)),
    out_shape=jax.ShapeDtypeStruct((VOCAB, 2 * DIM), jnp.float32),
    compiler_params=pltpu.CompilerParams(fuse_transposed_lhs_in_matmul=True),
)

_mesh = plsc.VectorSubcoreMesh(
    core_axis_name="c", subcore_axis_name="s", num_cores=NC, num_subcores=NS
)


@functools.partial(
    pl.kernel,
    out_type=jax.ShapeDtypeStruct((SEQ * NROW, 2 * DIM), jnp.float32),
    mesh=_mesh,
    scratch_types=[
        pltpu.VMEM((SEQ, NPW), jnp.int32),
        pltpu.VMEM((NBUF, NL, DIM), jnp.float32),
        [pltpu.SemaphoreType.DMA] * NBUF,
    ],
    compiler_params=pltpu.CompilerParams(use_tc_tiling_on_sc=False),
)
def _emb_gather(xt_hbm, table_hbm, out_hbm, idx_v, rows_v, gsems):
    wid = lax.axis_index("s") * NC + lax.axis_index("c")
    nbase = wid * NPW
    pltpu.sync_copy(xt_hbm.at[:, pl.ds(nbase, NPW)], idx_v)

    def fire(kk, b):
        s = kk // NB
        nb = kk % NB
        pltpu.async_copy(table_hbm.at[idx_v.at[s, pl.ds(nb * NL, NL)]],
                         rows_v.at[b], gsems[b])

    def drain(kk, b):
        s = kk // NB
        nb = kk % NB
        pltpu.make_async_copy(table_hbm.at[idx_v.at[s, pl.ds(nb * NL, NL)]],
                              rows_v.at[b], gsems[b]).wait()
        pltpu.sync_copy(
            rows_v.at[b],
            out_hbm.at[pl.ds(s * NROW + nbase + nb * NL, NL), pl.ds(0, DIM)])

    for b in range(NBUF):
        fire(b, b)

    @pl.loop(0, OUTER - 1)
    def _outer(o):
        for b in range(NBUF):
            kk = o * NBUF + b
            drain(kk, b)
            fire(kk + NBUF, b)

    for b in range(NBUF):
        drain((OUTER - 1) * NBUF + b, b)


_TN = 8192


def _fmt_body(g_ref, o_ref):
    e = jnp.eye(128, dtype=jnp.float32)
    for ci in range(_TN // 128):
        uc = g_ref[pl.ds(ci * 128, 128), :DIM]
        o_ref[0, :, pl.ds(ci * 128, 128)] = lax.dot_general(
            uc, e, (((0,), (0,)), ((), ())),
            preferred_element_type=jnp.float32,
            precision=lax.Precision.HIGHEST)


_tc_format = pl.pallas_call(
    _fmt_body,
    grid=(SEQ, NROW // _TN),
    in_specs=[pl.BlockSpec((_TN, 2 * DIM),
                           lambda s, nb: (s * (NROW // _TN) + nb, 0))],
    out_specs=pl.BlockSpec((1, DIM, _TN), lambda s, nb: (s, 0, nb)),
    out_shape=jax.ShapeDtypeStruct((SEQ, DIM, NROW), jnp.float32),
    compiler_params=pltpu.CompilerParams(fuse_transposed_lhs_in_matmul=True),
)


def kernel(x, emb):
    t2 = _tc_table(emb.T)
    table = t2.reshape(2 * VOCAB, DIM)
    g2 = _emb_gather((x * 2).T, table)
    ol = _tc_format(g2)
    return ol.transpose(2, 0, 1)


# speedup vs baseline: 3.0460x; 2.4097x over previous
import functools

import jax
import jax.numpy as jnp
from jax import lax
from jax.experimental import pallas as pl
from jax.experimental.pallas import tpu as pltpu
from jax.experimental.pallas import tpu_sc as plsc

VOCAB = 1_000_000
DIM = 64
SEQ = 50
NROW = 16384

NC = 2
NS = 16
NW = NC * NS

NPW = NROW // NW
NL = 256
NB = NPW // NL
K = SEQ * NB
NBUF = 4
OUTER = K // NBUF

_BT = 8192
_TGRID = (VOCAB + _BT - 1) // _BT


def _tt_body(a_ref, o_ref):
    a = a_ref[...]
    e = jnp.eye(DIM, dtype=jnp.float32)
    t = lax.dot_general(a, e, (((0,), (0,)), ((), ())),
                        preferred_element_type=jnp.float32,
                        precision=lax.Precision.HIGHEST)
    ep = jnp.concatenate([e, jnp.zeros((DIM, DIM), jnp.float32)], axis=1)
    o_ref[...] = lax.dot_general(t, ep, (((1,), (0,)), ((), ())),
                                 preferred_element_type=jnp.float32,
                                 precision=lax.Precision.HIGHEST)


_tc_table = pl.pallas_call(
    _tt_body,
    grid=(_TGRID,),
    in_specs=[pl.BlockSpec((DIM, _BT), lambda j: (0, j))],
    out_specs=pl.BlockSpec((_BT, 2 * DIM), lambda j: (j, 0)),
    out_shape=jax.ShapeDtypeStruct((VOCAB, 2 * DIM), jnp.float32),
    compiler_params=pltpu.CompilerParams(fuse_transposed_lhs_in_matmul=True),
)

_mesh = plsc.VectorSubcoreMesh(
    core_axis_name="c", subcore_axis_name="s", num_cores=NC, num_subcores=NS
)


@functools.partial(
    pl.kernel,
    out_type=jax.ShapeDtypeStruct((SEQ * NROW, 2 * DIM), jnp.float32),
    mesh=_mesh,
    scratch_types=[
        pltpu.VMEM((SEQ, NPW), jnp.int32),
        pltpu.VMEM((NBUF, NL, DIM), jnp.float32),
        [pltpu.SemaphoreType.DMA] * NBUF,
    ],
    compiler_params=pltpu.CompilerParams(use_tc_tiling_on_sc=False),
)
def _emb_gather(xt_hbm, table_hbm, out_hbm, idx_v, rows_v, gsems):
    wid = lax.axis_index("s") * NC + lax.axis_index("c")
    nbase = wid * NPW
    pltpu.sync_copy(xt_hbm.at[:, pl.ds(nbase, NPW)], idx_v)

    def fire(kk, b):
        s = kk // NB
        nb = kk % NB
        pltpu.async_copy(table_hbm.at[idx_v.at[s, pl.ds(nb * NL, NL)]],
                         rows_v.at[b], gsems[b])

    def drain(kk, b):
        s = kk // NB
        nb = kk % NB
        pltpu.make_async_copy(table_hbm.at[idx_v.at[s, pl.ds(nb * NL, NL)]],
                              rows_v.at[b], gsems[b]).wait()
        pltpu.sync_copy(
            rows_v.at[b],
            out_hbm.at[pl.ds(s * NROW + nbase + nb * NL, NL), pl.ds(0, DIM)])

    for b in range(NBUF):
        fire(b, b)

    @pl.loop(0, OUTER - 1)
    def _outer(o):
        for b in range(NBUF):
            kk = o * NBUF + b
            drain(kk, b)
            fire(kk + NBUF, b)

    for b in range(NBUF):
        drain((OUTER - 1) * NBUF + b, b)


_TN = 8192


def _fmt_body(g_ref, o_ref):
    e = jnp.eye(128, dtype=jnp.float32)
    for ci in range(_TN // 128):
        uc = g_ref[pl.ds(ci * 128, 128), :DIM]
        o_ref[0, :, pl.ds(ci * 128, 128)] = lax.dot_general(
            uc, e, (((0,), (0,)), ((), ())),
            preferred_element_type=jnp.float32,
            precision=lax.Precision.HIGHEST)


_tc_format = pl.pallas_call(
    _fmt_body,
    grid=(SEQ, NROW // _TN),
    in_specs=[pl.BlockSpec((_TN, 2 * DIM),
                           lambda s, nb: (s * (NROW // _TN) + nb, 0))],
    out_specs=pl.BlockSpec((1, DIM, _TN), lambda s, nb: (s, 0, nb)),
    out_shape=jax.ShapeDtypeStruct((SEQ, DIM, NROW), jnp.float32),
    compiler_params=pltpu.CompilerParams(fuse_transposed_lhs_in_matmul=True),
)


def kernel(x, emb):
    z = jnp.pad(emb, ((0, 0), (0, DIM)))
    table = z.reshape(2 * VOCAB, DIM)
    g2 = _emb_gather((x * 2).T, table)
    return g2[:, :DIM].reshape(SEQ, NROW, DIM).transpose(1, 0, 2)


# speedup vs baseline: 4.4605x; 1.4644x over previous
import functools

import jax
import jax.numpy as jnp
from jax import lax
from jax.experimental import pallas as pl
from jax.experimental.pallas import tpu as pltpu
from jax.experimental.pallas import tpu_sc as plsc

VOCAB = 1_000_000
DIM = 64
SEQ = 50
NROW = 16384

NC = 2
NS = 16
NW = NC * NS

NPW = NROW // NW
NL = 256
NB = NPW // NL
K = SEQ * NB
NBUF = 4
OUTER = K // NBUF

_BT = 8192
_TGRID = (VOCAB + _BT - 1) // _BT


def _tt_body(a_ref, o_ref):
    a = a_ref[...]
    o_ref[:, :DIM] = a.T
    o_ref[:, DIM:] = jnp.zeros((_BT, DIM), jnp.float32)


_tc_table = pl.pallas_call(
    _tt_body,
    grid=(_TGRID,),
    in_specs=[pl.BlockSpec((DIM, _BT), lambda j: (0, j))],
    out_specs=pl.BlockSpec((_BT, 2 * DIM), lambda j: (j, 0)),
    out_shape=jax.ShapeDtypeStruct((VOCAB, 2 * DIM), jnp.float32),
    compiler_params=pltpu.CompilerParams(fuse_transposed_lhs_in_matmul=True),
)

_mesh = plsc.VectorSubcoreMesh(
    core_axis_name="c", subcore_axis_name="s", num_cores=NC, num_subcores=NS
)


@functools.partial(
    pl.kernel,
    out_type=jax.ShapeDtypeStruct((SEQ * NROW, 2 * DIM), jnp.float32),
    mesh=_mesh,
    scratch_types=[
        pltpu.VMEM((SEQ, NPW), jnp.int32),
        pltpu.VMEM((NBUF, NL, DIM), jnp.float32),
        [pltpu.SemaphoreType.DMA] * NBUF,
    ],
    compiler_params=pltpu.CompilerParams(use_tc_tiling_on_sc=False),
)
def _emb_gather(xt_hbm, table_hbm, out_hbm, idx_v, rows_v, gsems):
    wid = lax.axis_index("s") * NC + lax.axis_index("c")
    nbase = wid * NPW
    pltpu.sync_copy(xt_hbm.at[:, pl.ds(nbase, NPW)], idx_v)

    def fire(kk, b):
        s = kk // NB
        nb = kk % NB
        pltpu.async_copy(table_hbm.at[idx_v.at[s, pl.ds(nb * NL, NL)]],
                         rows_v.at[b], gsems[b])

    def drain(kk, b):
        s = kk // NB
        nb = kk % NB
        pltpu.make_async_copy(table_hbm.at[idx_v.at[s, pl.ds(nb * NL, NL)]],
                              rows_v.at[b], gsems[b]).wait()
        pltpu.sync_copy(
            rows_v.at[b],
            out_hbm.at[pl.ds(s * NROW + nbase + nb * NL, NL), pl.ds(0, DIM)])

    for b in range(NBUF):
        fire(b, b)

    @pl.loop(0, OUTER - 1)
    def _outer(o):
        for b in range(NBUF):
            kk = o * NBUF + b
            drain(kk, b)
            fire(kk + NBUF, b)

    for b in range(NBUF):
        drain((OUTER - 1) * NBUF + b, b)


_TN = 8192


def _fmt_body(g_ref, o_ref):
    e = jnp.eye(128, dtype=jnp.float32)
    for ci in range(_TN // 128):
        uc = g_ref[pl.ds(ci * 128, 128), :DIM]
        o_ref[0, :, pl.ds(ci * 128, 128)] = lax.dot_general(
            uc, e, (((0,), (0,)), ((), ())),
            preferred_element_type=jnp.float32,
            precision=lax.Precision.HIGHEST)


_tc_format = pl.pallas_call(
    _fmt_body,
    grid=(SEQ, NROW // _TN),
    in_specs=[pl.BlockSpec((_TN, 2 * DIM),
                           lambda s, nb: (s * (NROW // _TN) + nb, 0))],
    out_specs=pl.BlockSpec((1, DIM, _TN), lambda s, nb: (s, 0, nb)),
    out_shape=jax.ShapeDtypeStruct((SEQ, DIM, NROW), jnp.float32),
    compiler_params=pltpu.CompilerParams(fuse_transposed_lhs_in_matmul=True),
)


def kernel(x, emb):
    z = _tc_table(emb.T)
    table = z.reshape(2 * VOCAB, DIM)
    g2 = _emb_gather((x * 2).T, table)
    return g2[:, :DIM].reshape(SEQ, NROW, DIM).transpose(1, 0, 2)


# speedup vs baseline: 4.4672x; 1.0015x over previous
import functools

import jax
import jax.numpy as jnp
from jax import lax
from jax.experimental import pallas as pl
from jax.experimental.pallas import tpu as pltpu
from jax.experimental.pallas import tpu_sc as plsc

VOCAB = 1_000_000
DIM = 64
SEQ = 50
NROW = 16384

NC = 2
NS = 16
NW = NC * NS

NPW = NROW // NW
NL = 256
NB = NPW // NL
K = SEQ * NB
NBUF = 4
OUTER = K // NBUF

_BT = 8192
_TGRID = (VOCAB + _BT - 1) // _BT


def _tt_body(a_ref, o_ref):
    a = a_ref[...]
    o_ref[:, :DIM] = a.T


_tc_table = pl.pallas_call(
    _tt_body,
    grid=(_TGRID,),
    in_specs=[pl.BlockSpec((DIM, _BT), lambda j: (0, j))],
    out_specs=pl.BlockSpec((_BT, 2 * DIM), lambda j: (j, 0)),
    out_shape=jax.ShapeDtypeStruct((VOCAB, 2 * DIM), jnp.float32),
    compiler_params=pltpu.CompilerParams(fuse_transposed_lhs_in_matmul=True),
)

_mesh = plsc.VectorSubcoreMesh(
    core_axis_name="c", subcore_axis_name="s", num_cores=NC, num_subcores=NS
)


@functools.partial(
    pl.kernel,
    out_type=jax.ShapeDtypeStruct((SEQ * NROW, 2 * DIM), jnp.float32),
    mesh=_mesh,
    scratch_types=[
        pltpu.VMEM((SEQ, NPW), jnp.int32),
        pltpu.VMEM((NBUF, NL, DIM), jnp.float32),
        [pltpu.SemaphoreType.DMA] * NBUF,
    ],
    compiler_params=pltpu.CompilerParams(use_tc_tiling_on_sc=False),
)
def _emb_gather(xt_hbm, table_hbm, out_hbm, idx_v, rows_v, gsems):
    wid = lax.axis_index("s") * NC + lax.axis_index("c")
    nbase = wid * NPW
    pltpu.sync_copy(xt_hbm.at[:, pl.ds(nbase, NPW)], idx_v)

    def fire(kk, b):
        s = kk // NB
        nb = kk % NB
        pltpu.async_copy(table_hbm.at[idx_v.at[s, pl.ds(nb * NL, NL)]],
                         rows_v.at[b], gsems[b])

    def drain(kk, b):
        s = kk // NB
        nb = kk % NB
        pltpu.make_async_copy(table_hbm.at[idx_v.at[s, pl.ds(nb * NL, NL)]],
                              rows_v.at[b], gsems[b]).wait()
        pltpu.sync_copy(
            rows_v.at[b],
            out_hbm.at[pl.ds(s * NROW + nbase + nb * NL, NL), pl.ds(0, DIM)])

    for b in range(NBUF):
        fire(b, b)

    @pl.loop(0, OUTER - 1)
    def _outer(o):
        for b in range(NBUF):
            kk = o * NBUF + b
            drain(kk, b)
            fire(kk + NBUF, b)

    for b in range(NBUF):
        drain((OUTER - 1) * NBUF + b, b)


_TN = 8192


def _fmt_body(g_ref, o_ref):
    e = jnp.eye(128, dtype=jnp.float32)
    for ci in range(_TN // 128):
        uc = g_ref[pl.ds(ci * 128, 128), :DIM]
        o_ref[0, :, pl.ds(ci * 128, 128)] = lax.dot_general(
            uc, e, (((0,), (0,)), ((), ())),
            preferred_element_type=jnp.float32,
            precision=lax.Precision.HIGHEST)


_tc_format = pl.pallas_call(
    _fmt_body,
    grid=(SEQ, NROW // _TN),
    in_specs=[pl.BlockSpec((_TN, 2 * DIM),
                           lambda s, nb: (s * (NROW // _TN) + nb, 0))],
    out_specs=pl.BlockSpec((1, DIM, _TN), lambda s, nb: (s, 0, nb)),
    out_shape=jax.ShapeDtypeStruct((SEQ, DIM, NROW), jnp.float32),
    compiler_params=pltpu.CompilerParams(fuse_transposed_lhs_in_matmul=True),
)


def kernel(x, emb):
    z = _tc_table(emb.T)
    table = z.reshape(2 * VOCAB, DIM)
    g2 = _emb_gather((x * 2).T, table)
    return g2[:, :DIM].reshape(SEQ, NROW, DIM).transpose(1, 0, 2)


# speedup vs baseline: 4.6202x; 1.0343x over previous
import functools

import jax
import jax.numpy as jnp
from jax import lax
from jax.experimental import pallas as pl
from jax.experimental.pallas import tpu as pltpu
from jax.experimental.pallas import tpu_sc as plsc

VOCAB = 1_000_000
DIM = 64
SEQ = 50
NROW = 16384

NC = 2
NS = 16
NW = NC * NS

NPW = NROW // NW
NL = 256
NB = NPW // NL
K = SEQ * NB
NBUF = 5
OUTER = K // NBUF

_BT = 16384
_TGRID = (VOCAB + _BT - 1) // _BT


def _tt_body(a_ref, o_ref):
    a = a_ref[...]
    o_ref[:, :DIM] = a.T


_tc_table = pl.pallas_call(
    _tt_body,
    grid=(_TGRID,),
    in_specs=[pl.BlockSpec((DIM, _BT), lambda j: (0, j))],
    out_specs=pl.BlockSpec((_BT, 2 * DIM), lambda j: (j, 0)),
    out_shape=jax.ShapeDtypeStruct((VOCAB, 2 * DIM), jnp.float32),
    compiler_params=pltpu.CompilerParams(fuse_transposed_lhs_in_matmul=True),
)

_mesh = plsc.VectorSubcoreMesh(
    core_axis_name="c", subcore_axis_name="s", num_cores=NC, num_subcores=NS
)


@functools.partial(
    pl.kernel,
    out_type=jax.ShapeDtypeStruct((SEQ * NROW, 2 * DIM), jnp.float32),
    mesh=_mesh,
    scratch_types=[
        pltpu.VMEM((SEQ, NPW), jnp.int32),
        pltpu.VMEM((NBUF, NL, DIM), jnp.float32),
        [pltpu.SemaphoreType.DMA] * NBUF,
    ],
    compiler_params=pltpu.CompilerParams(use_tc_tiling_on_sc=False),
)
def _emb_gather(xt_hbm, table_hbm, out_hbm, idx_v, rows_v, gsems):
    wid = lax.axis_index("s") * NC + lax.axis_index("c")
    nbase = wid * NPW
    pltpu.sync_copy(xt_hbm.at[:, pl.ds(nbase, NPW)], idx_v)

    def fire(kk, b):
        s = kk // NB
        nb = kk % NB
        pltpu.async_copy(table_hbm.at[idx_v.at[s, pl.ds(nb * NL, NL)]],
                         rows_v.at[b], gsems[b])

    def drain(kk, b):
        s = kk // NB
        nb = kk % NB
        pltpu.make_async_copy(table_hbm.at[idx_v.at[s, pl.ds(nb * NL, NL)]],
                              rows_v.at[b], gsems[b]).wait()
        pltpu.sync_copy(
            rows_v.at[b],
            out_hbm.at[pl.ds(s * NROW + nbase + nb * NL, NL), pl.ds(0, DIM)])

    for b in range(NBUF):
        fire(b, b)

    @pl.loop(0, OUTER - 1)
    def _outer(o):
        for b in range(NBUF):
            kk = o * NBUF + b
            drain(kk, b)
            fire(kk + NBUF, b)

    for b in range(NBUF):
        drain((OUTER - 1) * NBUF + b, b)


_TN = 8192


def _fmt_body(g_ref, o_ref):
    e = jnp.eye(128, dtype=jnp.float32)
    for ci in range(_TN // 128):
        uc = g_ref[pl.ds(ci * 128, 128), :DIM]
        o_ref[0, :, pl.ds(ci * 128, 128)] = lax.dot_general(
            uc, e, (((0,), (0,)), ((), ())),
            preferred_element_type=jnp.float32,
            precision=lax.Precision.HIGHEST)


_tc_format = pl.pallas_call(
    _fmt_body,
    grid=(SEQ, NROW // _TN),
    in_specs=[pl.BlockSpec((_TN, 2 * DIM),
                           lambda s, nb: (s * (NROW // _TN) + nb, 0))],
    out_specs=pl.BlockSpec((1, DIM, _TN), lambda s, nb: (s, 0, nb)),
    out_shape=jax.ShapeDtypeStruct((SEQ, DIM, NROW), jnp.float32),
    compiler_params=pltpu.CompilerParams(fuse_transposed_lhs_in_matmul=True),
)


def kernel(x, emb):
    z = _tc_table(emb.T)
    table = z.reshape(2 * VOCAB, DIM)
    g2 = _emb_gather((x * 2).T, table)
    return g2[:, :DIM].reshape(SEQ, NROW, DIM).transpose(1, 0, 2)


# speedup vs baseline: 4.6226x; 1.0005x over previous
import functools

import jax
import jax.numpy as jnp
from jax import lax
from jax.experimental import pallas as pl
from jax.experimental.pallas import tpu as pltpu
from jax.experimental.pallas import tpu_sc as plsc

VOCAB = 1_000_000
DIM = 64
SEQ = 50
NROW = 16384

NC = 2
NS = 16
NW = NC * NS

NPW = NROW // NW
NL = 256
NB = NPW // NL
K = SEQ * NB
NBUF = 5
OUTER = K // NBUF

_BT = 16384
_TGRID = (VOCAB + _BT - 1) // _BT


def _tt_body(a_ref, o_ref):
    a = a_ref[...]
    o_ref[:, :DIM] = a.T


_tc_table = pl.pallas_call(
    _tt_body,
    grid=(_TGRID,),
    in_specs=[pl.BlockSpec((DIM, _BT), lambda j: (0, j))],
    out_specs=pl.BlockSpec((_BT, 2 * DIM), lambda j: (j, 0)),
    out_shape=jax.ShapeDtypeStruct((VOCAB, 2 * DIM), jnp.float32),
)

_mesh = plsc.VectorSubcoreMesh(
    core_axis_name="c", subcore_axis_name="s", num_cores=NC, num_subcores=NS
)


@functools.partial(
    pl.kernel,
    out_type=jax.ShapeDtypeStruct((SEQ * NROW, 2 * DIM), jnp.float32),
    mesh=_mesh,
    scratch_types=[
        pltpu.VMEM((SEQ, NPW), jnp.int32),
        pltpu.VMEM((NBUF, NL, DIM), jnp.float32),
        [pltpu.SemaphoreType.DMA] * NBUF,
    ],
    compiler_params=pltpu.CompilerParams(use_tc_tiling_on_sc=False),
)
def _emb_gather(xt_hbm, table_hbm, out_hbm, idx_v, rows_v, gsems):
    wid = lax.axis_index("s") * NC + lax.axis_index("c")
    nbase = wid * NPW
    pltpu.sync_copy(xt_hbm.at[:, pl.ds(nbase, NPW)], idx_v)

    def fire(kk, b):
        s = kk // NB
        nb = kk % NB
        pltpu.async_copy(table_hbm.at[idx_v.at[s, pl.ds(nb * NL, NL)]],
                         rows_v.at[b], gsems[b])

    def drain(kk, b):
        s = kk // NB
        nb = kk % NB
        pltpu.make_async_copy(table_hbm.at[idx_v.at[s, pl.ds(nb * NL, NL)]],
                              rows_v.at[b], gsems[b]).wait()
        pltpu.sync_copy(
            rows_v.at[b],
            out_hbm.at[pl.ds(s * NROW + nbase + nb * NL, NL), pl.ds(0, DIM)])

    for b in range(NBUF):
        fire(b, b)

    @pl.loop(0, OUTER - 1)
    def _outer(o):
        for b in range(NBUF):
            kk = o * NBUF + b
            drain(kk, b)
            fire(kk + NBUF, b)

    for b in range(NBUF):
        drain((OUTER - 1) * NBUF + b, b)


def kernel(x, emb):
    z = _tc_table(emb.T)
    table = z.reshape(2 * VOCAB, DIM)
    g2 = _emb_gather((x * 2).T, table)
    return g2[:, :DIM].reshape(SEQ, NROW, DIM).transpose(1, 0, 2)
